# baseline (speedup 1.0000x reference)
import functools

import jax
import jax.numpy as jnp
from jax import lax
from jax.experimental import pallas as pl
from jax.experimental.pallas import tpu as pltpu
from jax.experimental.pallas import tpu_sc as plsc

D = 64
L = 16
IDX_CHUNK = 128


def _recommender_scores(user_ids, item_ids, user_table, item_table, *,
                        n_workers, b_per_w):
    n_chunks = b_per_w // IDX_CHUNK
    mesh = plsc.VectorSubcoreMesh(core_axis_name="c", subcore_axis_name="s")

    @functools.partial(
        pl.kernel,
        mesh=mesh,
        compiler_params=pltpu.CompilerParams(needs_layout_passes=False,
                                             use_tc_tiling_on_sc=False),
        out_type=jax.ShapeDtypeStruct((n_workers, b_per_w), jnp.float32),
        scratch_types=[
            pltpu.VMEM((n_chunks, IDX_CHUNK), jnp.int32),
            pltpu.VMEM((n_chunks, IDX_CHUNK), jnp.int32),
            pltpu.VMEM((b_per_w, D), jnp.float32),
            pltpu.VMEM((b_per_w, D), jnp.float32),
            pltpu.VMEM((L * b_per_w,), jnp.float32),
            pltpu.VMEM((b_per_w,), jnp.float32),
            pltpu.SemaphoreType.DMA,
            pltpu.SemaphoreType.DMA,
        ],
    )
    def k(uid_hbm, iid_hbm, utab_hbm, itab_hbm, out_hbm,
          uidx_v, iidx_v, urows_v, irows_v, tpose_v, out_v, usem, isem):
        wid = lax.axis_index("s") * mesh.num_cores + lax.axis_index("c")
        pltpu.sync_copy(uid_hbm.at[wid], uidx_v)
        pltpu.sync_copy(iid_hbm.at[wid], iidx_v)
        for j in range(n_chunks):
            pltpu.async_copy(utab_hbm.at[uidx_v.at[j]],
                             urows_v.at[pl.ds(j * IDX_CHUNK, IDX_CHUNK)], usem)
            pltpu.async_copy(itab_hbm.at[iidx_v.at[j]],
                             irows_v.at[pl.ds(j * IDX_CHUNK, IDX_CHUNK)], isem)
        for j in range(n_chunks):
            pltpu.make_async_copy(utab_hbm.at[uidx_v.at[j]],
                                  urows_v.at[pl.ds(j * IDX_CHUNK, IDX_CHUNK)],
                                  usem).wait()
            pltpu.make_async_copy(itab_hbm.at[iidx_v.at[j]],
                                  irows_v.at[pl.ds(j * IDX_CHUNK, IDX_CHUNK)],
                                  isem).wait()

        lane_ids = lax.iota(jnp.int32, L)

        @plsc.parallel_loop(0, b_per_w, 1, unroll=8)
        def body(b):
            acc = urows_v[b, pl.ds(0, L)] * irows_v[b, pl.ds(0, L)]
            for c in range(1, D // L):
                acc += urows_v[b, pl.ds(c * L, L)] * irows_v[b, pl.ds(c * L, L)]
            flat_idx = lane_ids * b_per_w + b
            plsc.store_scatter(tpose_v, [flat_idx], acc)

        @plsc.parallel_loop(0, b_per_w // L, 1, unroll=2)
        def reduce_body(m):
            acc = tpose_v[pl.ds(m * L, L)]
            for c in range(1, L):
                acc += tpose_v[pl.ds(c * b_per_w + m * L, L)]
            out_v[pl.ds(m * L, L)] = acc

        pltpu.sync_copy(out_v, out_hbm.at[wid])

    return k(user_ids, item_ids, user_table, item_table)


def kernel(user_ids, item_ids, user_table, item_table):
    b = user_ids.shape[0]
    info = plsc.get_sparse_core_info()
    n_workers = info.num_cores * info.num_subcores
    b_per_w = b // n_workers
    uid = user_ids.astype(jnp.int32).reshape(n_workers, b_per_w // IDX_CHUNK,
                                             IDX_CHUNK)
    iid = item_ids.astype(jnp.int32).reshape(n_workers, b_per_w // IDX_CHUNK,
                                             IDX_CHUNK)
    out = _recommender_scores(uid, iid, user_table, item_table,
                              n_workers=n_workers, b_per_w=b_per_w)
    return out.reshape(b)
